# Initial kernel scaffold; baseline (speedup 1.0000x reference)
import jax
import jax.numpy as jnp
from jax.experimental import pallas as pl


def kernel(user_idx, context_idx, edge_index, user_emb, service_emb, gcn1_W, gcn1_b, gcn2_W, gcn2_b, fc_W, fc_b):
    raise NotImplementedError("write your pallas kernel here")


# speedup vs baseline: 31.1694x; 31.1694x over previous
import functools

import jax
import jax.numpy as jnp
from jax import lax
from jax.experimental import pallas as pl
from jax.experimental.pallas import tpu as pltpu
from jax.experimental.pallas import tpu_sc as plsc

N_SERVICES = 50000
N_USERS = 100000
D = 32
BATCH = 1024
CTX = 3
N_EDGES = 1600000

NP = 50048
NR = 12500
NRP = NP // 4
STRIPE = NP // 16
EPC = 12544
CPT = EPC // 32
SUP = 8
OUTER = CPT // SUP
U_PER = BATCH // 32
C_PER = (BATCH * CTX) // 32

_mesh = plsc.VectorSubcoreMesh(core_axis_name="c", subcore_axis_name="s")
_sc_params = pltpu.CompilerParams(use_tc_tiling_on_sc=False)

_NT = (((1,), (1,)), ((), ()))


@functools.partial(
    pl.kernel,
    out_type=[
        jax.ShapeDtypeStruct((2 * NP,), jnp.float32),
        jax.ShapeDtypeStruct((BATCH, D), jnp.float32),
    ],
    mesh=_mesh,
    scratch_types=[
        pltpu.VMEM_SHARED((NP,), jnp.float32),
        pltpu.VMEM((SUP, 128), jnp.int32),
        pltpu.VMEM((128,), jnp.float32),
        pltpu.VMEM((U_PER,), jnp.int32),
        pltpu.VMEM((U_PER, D), jnp.float32),
        pltpu.SemaphoreType.DMA,
    ],
    compiler_params=_sc_params,
)
def _sc_deg_user(col_hbm, uidx_hbm, uemb_hbm, z1_hbm,
                 deg_hbm, uvec_hbm,
                 deg_sh, colb, onesb, uidxb, urows, sem):
    c = lax.axis_index("c")
    s = lax.axis_index("s")
    wid = s * 2 + c

    pltpu.sync_copy(uidx_hbm.at[pl.ds(wid * U_PER, U_PER)], uidxb)
    pltpu.async_copy(uemb_hbm.at[uidxb], urows, sem).wait()
    pltpu.sync_copy(urows, uvec_hbm.at[pl.ds(wid * U_PER, U_PER)])

    pltpu.sync_copy(z1_hbm.at[pl.ds(s * STRIPE, STRIPE)],
                    deg_sh.at[pl.ds(s * STRIPE, STRIPE)])
    for i in range(8):
        onesb[pl.ds(i * 16, 16)] = jnp.full((16,), 1.0, jnp.float32)
    plsc.subcore_barrier()

    def outer(o, carry):
        base = wid * CPT + o * SUP
        pltpu.sync_copy(col_hbm.at[pl.ds(base, SUP)], colb)
        for j in range(SUP):
            pltpu.sync_copy(onesb, deg_sh.at[colb.at[j]], add=True)
        return carry

    lax.fori_loop(0, OUTER, outer, 0)
    plsc.subcore_barrier()
    pltpu.sync_copy(deg_sh.at[pl.ds(s * STRIPE, STRIPE)],
                    deg_hbm.at[pl.ds(c * NP + s * STRIPE, STRIPE)])


@functools.partial(
    pl.kernel,
    out_type=jax.ShapeDtypeStruct((2, NP, D), jnp.float32),
    mesh=_mesh,
    scratch_types=[
        pltpu.VMEM_SHARED((NP, D), jnp.float32),
        pltpu.VMEM((SUP, 128), jnp.int32),
        pltpu.VMEM((SUP, 128), jnp.int32),
        pltpu.VMEM((2, 128, D), jnp.float32),
        pltpu.SemaphoreType.DMA((2,)),
        pltpu.SemaphoreType.DMA((2,)),
    ],
    compiler_params=_sc_params,
)
def _sc_spmm(y_hbm, row_hbm, col_hbm, z2_hbm,
             acc_hbm,
             acc_sh, rowb, colb, gbuf, gsem, ssem):
    c = lax.axis_index("c")
    s = lax.axis_index("s")
    wid = s * 2 + c

    pltpu.sync_copy(z2_hbm.at[pl.ds(s * STRIPE, STRIPE)],
                    acc_sh.at[pl.ds(s * STRIPE, STRIPE)])
    plsc.subcore_barrier()

    def outer(o, carry):
        base = wid * CPT + o * SUP
        pltpu.sync_copy(row_hbm.at[pl.ds(base, SUP)], rowb)
        pltpu.sync_copy(col_hbm.at[pl.ds(base, SUP)], colb)
        for j in range(SUP):
            pltpu.async_copy(y_hbm.at[rowb.at[j]], gbuf.at[0], gsem.at[0]).wait()
            pltpu.sync_copy(gbuf.at[0], acc_sh.at[colb.at[j]], add=True)
        return carry

    lax.fori_loop(0, OUTER, outer, 0)
    plsc.subcore_barrier()
    pltpu.sync_copy(acc_sh.at[pl.ds(s * STRIPE, STRIPE)],
                    acc_hbm.at[c, pl.ds(s * STRIPE, STRIPE)])


@functools.partial(
    pl.kernel,
    out_type=jax.ShapeDtypeStruct((BATCH * CTX, D), jnp.float32),
    mesh=_mesh,
    scratch_types=[
        pltpu.VMEM((C_PER,), jnp.int32),
        pltpu.VMEM((C_PER, D), jnp.float32),
        pltpu.SemaphoreType.DMA,
    ],
    compiler_params=_sc_params,
)
def _sc_ctx_gather(h2_hbm, cidx_hbm, out_hbm, cidxb, crows, sem):
    c = lax.axis_index("c")
    s = lax.axis_index("s")
    wid = s * 2 + c
    pltpu.sync_copy(cidx_hbm.at[pl.ds(wid * C_PER, C_PER)], cidxb)
    pltpu.async_copy(h2_hbm.at[cidxb], crows, sem).wait()
    pltpu.sync_copy(crows, out_hbm.at[pl.ds(wid * C_PER, C_PER)])


def _tc_prep_body(emb_ref, deg_ref, w4_ref, bexp_ref, y1_ref, dinv_ref):
    d = deg_ref[0, :NR] + deg_ref[1, :NR] + 1.0
    dinv_n = lax.rsqrt(d)
    dinv = lax.dot_general(dinv_n, bexp_ref[...], (((1,), (0,)), ((), ())),
                           preferred_element_type=jnp.float32)
    dinv_ref[...] = dinv
    xw = lax.dot_general(emb_ref[...], w4_ref[...], (((1,), (0,)), ((), ())),
                         preferred_element_type=jnp.float32)
    y1_ref[:NR] = dinv * xw
    y1_ref[NR:] = jnp.zeros((NRP - NR, 128), jnp.float32)


_tc_prep = pl.pallas_call(
    _tc_prep_body,
    out_shape=[
        jax.ShapeDtypeStruct((NRP, 128), jnp.float32),
        jax.ShapeDtypeStruct((NR, 128), jnp.float32),
    ],
)


def _tc_mid_body(acc_ref, y1_ref, dinv_ref, b1_ref, w4_ref, y2_ref):
    z = acc_ref[0, :NR] + acc_ref[1, :NR] + y1_ref[:NR]
    dinv = dinv_ref[...]
    h1 = jnp.maximum(dinv * z + b1_ref[...], 0.0)
    y2 = dinv * lax.dot_general(h1, w4_ref[...], (((1,), (0,)), ((), ())),
                                preferred_element_type=jnp.float32)
    y2_ref[:NR] = y2
    y2_ref[NR:] = jnp.zeros((NRP - NR, 128), jnp.float32)


_tc_mid = pl.pallas_call(
    _tc_mid_body,
    out_shape=jax.ShapeDtypeStruct((NRP, 128), jnp.float32),
)


def _tc_post_body(acc_ref, y2_ref, dinv_ref, b2_ref, h2_ref):
    z = acc_ref[0, :NR] + acc_ref[1, :NR] + y2_ref[:NR]
    h2_ref[...] = dinv_ref[...] * z + b2_ref[...]


_tc_post = pl.pallas_call(
    _tc_post_body,
    out_shape=jax.ShapeDtypeStruct((NR, 128), jnp.float32),
)


SB = 1024


def _tc_fc_body(x_ref, w_ref, b_ref, out_ref):
    out_ref[...] = lax.dot_general(
        x_ref[...], w_ref[...], _NT,
        preferred_element_type=jnp.float32) + b_ref[...]


_tc_fc = pl.pallas_call(
    _tc_fc_body,
    grid=(pl.cdiv(N_SERVICES, SB),),
    in_specs=[
        pl.BlockSpec((BATCH, 128), lambda i: (0, 0)),
        pl.BlockSpec((SB, 128), lambda i: (i, 0)),
        pl.BlockSpec((1, SB), lambda i: (0, i)),
    ],
    out_specs=pl.BlockSpec((BATCH, SB), lambda i: (0, i)),
    out_shape=jax.ShapeDtypeStruct((BATCH, N_SERVICES), jnp.float32),
)


def kernel(user_idx, context_idx, edge_index, user_emb, service_emb,
           gcn1_W, gcn1_b, gcn2_W, gcn2_b, fc_W, fc_b):
    row = edge_index[0].astype(jnp.int32)
    col = edge_index[1].astype(jnp.int32)
    pad = jnp.full((EPC * 128 - N_EDGES,), N_SERVICES, jnp.int32)
    row2 = jnp.concatenate([row, pad]).reshape(EPC, 128)
    col2 = jnp.concatenate([col, pad]).reshape(EPC, 128)
    z1 = jnp.zeros((NP,), jnp.float32)
    z2 = jnp.zeros((NP, D), jnp.float32)

    zblk = jnp.zeros((D, D), jnp.float32)
    w4_1 = jnp.block([[gcn1_W.T if i == j else zblk for j in range(4)]
                      for i in range(4)])
    w4_2 = jnp.block([[gcn2_W.T if i == j else zblk for j in range(4)]
                      for i in range(4)])
    bexp = jnp.repeat(jnp.eye(4, dtype=jnp.float32), D, axis=1)
    b1_4 = jnp.tile(gcn1_b, 4).reshape(1, 128)
    b2_4 = jnp.tile(gcn2_b, 4).reshape(1, 128)

    degf, uvec = _sc_deg_user(col2, user_idx.astype(jnp.int32), user_emb, z1)
    y1p, dinvp = _tc_prep(service_emb.reshape(NR, 128),
                          degf.reshape(2, NRP, 4), w4_1, bexp)
    acc1 = _sc_spmm(y1p.reshape(NP, D), row2, col2, z2)
    y2p = _tc_mid(acc1.reshape(2, NRP, 128), y1p, dinvp, b1_4, w4_2)
    acc2 = _sc_spmm(y2p.reshape(NP, D), row2, col2, z2)
    h2p = _tc_post(acc2.reshape(2, NRP, 128), y2p, dinvp, b2_4)
    ctx = _sc_ctx_gather(h2p.reshape(N_SERVICES, D),
                         context_idx.reshape(-1).astype(jnp.int32))
    x = jnp.concatenate([uvec, ctx.reshape(BATCH, CTX * D)], axis=1)
    return _tc_fc(x, fc_W, fc_b.reshape(1, N_SERVICES))


# speedup vs baseline: 47.9896x; 1.5396x over previous
import functools

import jax
import jax.numpy as jnp
from jax import lax
from jax.experimental import pallas as pl
from jax.experimental.pallas import tpu as pltpu
from jax.experimental.pallas import tpu_sc as plsc

N_SERVICES = 50000
N_USERS = 100000
D = 32
BATCH = 1024
CTX = 3
N_EDGES = 1600000

NP = 50048
NR = 12500
NRP = NP // 4
STRIPE = NP // 16
EPC = 13440
CPT = EPC // 32
SUP = 3
OUTER = CPT // SUP
U_PER = BATCH // 32
C_PER = (BATCH * CTX) // 32

_mesh = plsc.VectorSubcoreMesh(core_axis_name="c", subcore_axis_name="s")
_sc_params = pltpu.CompilerParams(use_tc_tiling_on_sc=False)

_NT = (((1,), (1,)), ((), ()))


@functools.partial(
    pl.kernel,
    out_type=[
        jax.ShapeDtypeStruct((2 * NP,), jnp.float32),
        jax.ShapeDtypeStruct((BATCH, D), jnp.float32),
    ],
    mesh=_mesh,
    scratch_types=[
        pltpu.VMEM_SHARED((NP,), jnp.float32),
        pltpu.VMEM((SUP, 128), jnp.int32),
        pltpu.VMEM((128,), jnp.float32),
        pltpu.VMEM((U_PER,), jnp.int32),
        pltpu.VMEM((U_PER, D), jnp.float32),
        pltpu.SemaphoreType.DMA,
    ],
    compiler_params=_sc_params,
)
def _sc_deg_user(col_hbm, uidx_hbm, uemb_hbm, z1_hbm,
                 deg_hbm, uvec_hbm,
                 deg_sh, colb, onesb, uidxb, urows, sem):
    c = lax.axis_index("c")
    s = lax.axis_index("s")
    wid = s * 2 + c

    pltpu.sync_copy(uidx_hbm.at[pl.ds(wid * U_PER, U_PER)], uidxb)
    pltpu.async_copy(uemb_hbm.at[uidxb], urows, sem).wait()
    pltpu.sync_copy(urows, uvec_hbm.at[pl.ds(wid * U_PER, U_PER)])

    pltpu.sync_copy(z1_hbm.at[pl.ds(s * STRIPE, STRIPE)],
                    deg_sh.at[pl.ds(s * STRIPE, STRIPE)])
    for i in range(8):
        onesb[pl.ds(i * 16, 16)] = jnp.full((16,), 1.0, jnp.float32)
    plsc.subcore_barrier()

    def outer(o, carry):
        base = wid * CPT + o * SUP
        pltpu.sync_copy(col_hbm.at[pl.ds(base, SUP)], colb)
        for j in range(SUP):
            pltpu.async_copy(onesb, deg_sh.at[colb.at[j]], sem, add=True)
        for j in range(SUP):
            pltpu.make_async_copy(onesb, deg_sh.at[colb.at[j]], sem).wait()
        return carry

    lax.fori_loop(0, OUTER, outer, 0)
    plsc.subcore_barrier()
    pltpu.sync_copy(deg_sh.at[pl.ds(s * STRIPE, STRIPE)],
                    deg_hbm.at[pl.ds(c * NP + s * STRIPE, STRIPE)])


@functools.partial(
    pl.kernel,
    out_type=jax.ShapeDtypeStruct((2, NP, D), jnp.float32),
    mesh=_mesh,
    scratch_types=[
        pltpu.VMEM_SHARED((NP, D), jnp.float32),
        pltpu.VMEM((4, SUP, 128), jnp.int32),
        pltpu.VMEM((4, SUP, 128), jnp.int32),
        pltpu.VMEM((2, SUP, 128, D), jnp.float32),
        pltpu.SemaphoreType.DMA((4,)),
        pltpu.SemaphoreType.DMA((2,)),
        pltpu.SemaphoreType.DMA((2,)),
    ],
    compiler_params=_sc_params,
)
def _sc_spmm(y_hbm, row_hbm, col_hbm, z2_hbm,
             acc_hbm,
             acc_sh, ibr, ibc, gbuf, sem_i, sem_g, sem_s):
    c = lax.axis_index("c")
    s = lax.axis_index("s")
    wid = s * 2 + c
    base = wid * CPT

    pltpu.sync_copy(z2_hbm.at[pl.ds(s * STRIPE, STRIPE)],
                    acc_sh.at[pl.ds(s * STRIPE, STRIPE)])
    plsc.subcore_barrier()

    def fire_idx(g, d):
        off = base + g * SUP
        pltpu.async_copy(row_hbm.at[pl.ds(off, SUP)], ibr.at[d], sem_i.at[d])
        pltpu.async_copy(col_hbm.at[pl.ds(off, SUP)], ibc.at[d], sem_i.at[d])

    def wait_idx(g, d):
        off = base + g * SUP
        pltpu.make_async_copy(row_hbm.at[pl.ds(off, SUP)], ibr.at[d],
                              sem_i.at[d]).wait()
        pltpu.make_async_copy(col_hbm.at[pl.ds(off, SUP)], ibc.at[d],
                              sem_i.at[d]).wait()

    def fire_g(d, par):
        for j in range(SUP):
            pltpu.async_copy(y_hbm.at[ibr.at[d, j]], gbuf.at[par, j],
                             sem_g.at[par])

    def wait_g(d, par):
        for j in range(SUP):
            pltpu.make_async_copy(y_hbm.at[ibr.at[d, j]], gbuf.at[par, j],
                                  sem_g.at[par]).wait()

    def fire_s(d, par):
        for j in range(SUP):
            pltpu.async_copy(gbuf.at[par, j], acc_sh.at[ibc.at[d, j]],
                             sem_s.at[par], add=True)

    def wait_s(d, par):
        for j in range(SUP):
            pltpu.make_async_copy(gbuf.at[par, j], acc_sh.at[ibc.at[d, j]],
                                  sem_s.at[par]).wait()

    fire_idx(0, 0)
    fire_idx(1, 1)
    wait_idx(0, 0)
    fire_g(0, 0)

    def body(i, carry):
        for k in range(4):
            g = 4 * i + k
            par = k % 2
            q = 1 - par
            wait_g(k, par)
            fire_s(k, par)

            @pl.when(g >= 1)
            def _():
                wait_s((k - 1) % 4, q)

            @pl.when(g + 2 < OUTER)
            def _():
                fire_idx(g + 2, (k + 2) % 4)

            @pl.when(g + 1 < OUTER)
            def _():
                wait_idx(g + 1, (k + 1) % 4)
                fire_g((k + 1) % 4, q)
        return carry

    lax.fori_loop(0, OUTER // 4, body, 0)
    wait_s(3, 1)
    plsc.subcore_barrier()
    pltpu.sync_copy(acc_sh.at[pl.ds(s * STRIPE, STRIPE)],
                    acc_hbm.at[c, pl.ds(s * STRIPE, STRIPE)])


@functools.partial(
    pl.kernel,
    out_type=jax.ShapeDtypeStruct((BATCH * CTX, D), jnp.float32),
    mesh=_mesh,
    scratch_types=[
        pltpu.VMEM((C_PER,), jnp.int32),
        pltpu.VMEM((C_PER, D), jnp.float32),
        pltpu.SemaphoreType.DMA,
    ],
    compiler_params=_sc_params,
)
def _sc_ctx_gather(h2_hbm, cidx_hbm, out_hbm, cidxb, crows, sem):
    c = lax.axis_index("c")
    s = lax.axis_index("s")
    wid = s * 2 + c
    pltpu.sync_copy(cidx_hbm.at[pl.ds(wid * C_PER, C_PER)], cidxb)
    pltpu.async_copy(h2_hbm.at[cidxb], crows, sem).wait()
    pltpu.sync_copy(crows, out_hbm.at[pl.ds(wid * C_PER, C_PER)])


def _tc_prep_body(emb_ref, deg_ref, w4_ref, bexp_ref, y1_ref, dinv_ref):
    d = deg_ref[0, :NR] + deg_ref[1, :NR] + 1.0
    dinv_n = lax.rsqrt(d)
    dinv = lax.dot_general(dinv_n, bexp_ref[...], (((1,), (0,)), ((), ())),
                           preferred_element_type=jnp.float32)
    dinv_ref[...] = dinv
    xw = lax.dot_general(emb_ref[...], w4_ref[...], (((1,), (0,)), ((), ())),
                         preferred_element_type=jnp.float32)
    y1_ref[:NR] = dinv * xw
    y1_ref[NR:] = jnp.zeros((NRP - NR, 128), jnp.float32)


_tc_prep = pl.pallas_call(
    _tc_prep_body,
    out_shape=[
        jax.ShapeDtypeStruct((NRP, 128), jnp.float32),
        jax.ShapeDtypeStruct((NR, 128), jnp.float32),
    ],
)


def _tc_mid_body(acc_ref, y1_ref, dinv_ref, b1_ref, w4_ref, y2_ref):
    z = acc_ref[0, :NR] + acc_ref[1, :NR] + y1_ref[:NR]
    dinv = dinv_ref[...]
    h1 = jnp.maximum(dinv * z + b1_ref[...], 0.0)
    y2 = dinv * lax.dot_general(h1, w4_ref[...], (((1,), (0,)), ((), ())),
                                preferred_element_type=jnp.float32)
    y2_ref[:NR] = y2
    y2_ref[NR:] = jnp.zeros((NRP - NR, 128), jnp.float32)


_tc_mid = pl.pallas_call(
    _tc_mid_body,
    out_shape=jax.ShapeDtypeStruct((NRP, 128), jnp.float32),
)


def _tc_post_body(acc_ref, y2_ref, dinv_ref, b2_ref, h2_ref):
    z = acc_ref[0, :NR] + acc_ref[1, :NR] + y2_ref[:NR]
    h2_ref[...] = dinv_ref[...] * z + b2_ref[...]


_tc_post = pl.pallas_call(
    _tc_post_body,
    out_shape=jax.ShapeDtypeStruct((NR, 128), jnp.float32),
)


SB = 1024


def _tc_fc_body(x_ref, w_ref, b_ref, out_ref):
    out_ref[...] = lax.dot_general(
        x_ref[...], w_ref[...], _NT,
        preferred_element_type=jnp.float32) + b_ref[...]


_tc_fc = pl.pallas_call(
    _tc_fc_body,
    grid=(pl.cdiv(N_SERVICES, SB),),
    in_specs=[
        pl.BlockSpec((BATCH, 128), lambda i: (0, 0)),
        pl.BlockSpec((SB, 128), lambda i: (i, 0)),
        pl.BlockSpec((1, SB), lambda i: (0, i)),
    ],
    out_specs=pl.BlockSpec((BATCH, SB), lambda i: (0, i)),
    out_shape=jax.ShapeDtypeStruct((BATCH, N_SERVICES), jnp.float32),
)


def kernel(user_idx, context_idx, edge_index, user_emb, service_emb,
           gcn1_W, gcn1_b, gcn2_W, gcn2_b, fc_W, fc_b):
    row = edge_index[0].astype(jnp.int32)
    col = edge_index[1].astype(jnp.int32)
    pad = N_SERVICES + (jnp.arange(EPC * 128 - N_EDGES, dtype=jnp.int32)
                        % (NP - N_SERVICES))
    row2 = jnp.concatenate([row, pad]).reshape(EPC, 128)
    col2 = jnp.concatenate([col, pad]).reshape(EPC, 128)
    z1 = jnp.zeros((NP,), jnp.float32)
    z2 = jnp.zeros((NP, D), jnp.float32)

    zblk = jnp.zeros((D, D), jnp.float32)
    w4_1 = jnp.block([[gcn1_W.T if i == j else zblk for j in range(4)]
                      for i in range(4)])
    w4_2 = jnp.block([[gcn2_W.T if i == j else zblk for j in range(4)]
                      for i in range(4)])
    bexp = jnp.repeat(jnp.eye(4, dtype=jnp.float32), D, axis=1)
    b1_4 = jnp.tile(gcn1_b, 4).reshape(1, 128)
    b2_4 = jnp.tile(gcn2_b, 4).reshape(1, 128)

    degf, uvec = _sc_deg_user(col2, user_idx.astype(jnp.int32), user_emb, z1)
    y1p, dinvp = _tc_prep(service_emb.reshape(NR, 128),
                          degf.reshape(2, NRP, 4), w4_1, bexp)
    acc1 = _sc_spmm(y1p.reshape(NP, D), row2, col2, z2)
    y2p = _tc_mid(acc1.reshape(2, NRP, 128), y1p, dinvp, b1_4, w4_2)
    acc2 = _sc_spmm(y2p.reshape(NP, D), row2, col2, z2)
    h2p = _tc_post(acc2.reshape(2, NRP, 128), y2p, dinvp, b2_4)
    ctx = _sc_ctx_gather(h2p.reshape(N_SERVICES, D),
                         context_idx.reshape(-1).astype(jnp.int32))
    x = jnp.concatenate([uvec, ctx.reshape(BATCH, CTX * D)], axis=1)
    return _tc_fc(x, fc_W, fc_b.reshape(1, N_SERVICES))


# speedup vs baseline: 51.9693x; 1.0829x over previous
import functools

import jax
import jax.numpy as jnp
from jax import lax
from jax.experimental import pallas as pl
from jax.experimental.pallas import tpu as pltpu
from jax.experimental.pallas import tpu_sc as plsc

N_SERVICES = 50000
N_USERS = 100000
D = 32
BATCH = 1024
CTX = 3
N_EDGES = 1600000

NP = 50048
NR = 12500
NRP = NP // 4
STRIPE = NP // 16
EPC = 13440
CPT = EPC // 32
SUP = 3
OUTER = CPT // SUP
DSUP = 20
DOUTER = CPT // DSUP
U_PER = BATCH // 32
C_PER = (BATCH * CTX) // 32

_mesh = plsc.VectorSubcoreMesh(core_axis_name="c", subcore_axis_name="s")
_sc_params = pltpu.CompilerParams(use_tc_tiling_on_sc=False)

_NT = (((1,), (1,)), ((), ()))


@functools.partial(
    pl.kernel,
    out_type=[
        jax.ShapeDtypeStruct((2 * NP,), jnp.float32),
        jax.ShapeDtypeStruct((BATCH, D), jnp.float32),
    ],
    mesh=_mesh,
    scratch_types=[
        pltpu.VMEM_SHARED((NP,), jnp.float32),
        pltpu.VMEM((DSUP, 128), jnp.int32),
        pltpu.VMEM((128,), jnp.float32),
        pltpu.VMEM((U_PER,), jnp.int32),
        pltpu.VMEM((U_PER, D), jnp.float32),
        pltpu.SemaphoreType.DMA,
    ],
    compiler_params=_sc_params,
)
def _sc_deg_user(col_hbm, uidx_hbm, uemb_hbm, z1_hbm,
                 deg_hbm, uvec_hbm,
                 deg_sh, colb, onesb, uidxb, urows, sem):
    c = lax.axis_index("c")
    s = lax.axis_index("s")
    wid = s * 2 + c

    pltpu.sync_copy(uidx_hbm.at[pl.ds(wid * U_PER, U_PER)], uidxb)
    pltpu.async_copy(uemb_hbm.at[uidxb], urows, sem).wait()
    pltpu.sync_copy(urows, uvec_hbm.at[pl.ds(wid * U_PER, U_PER)])

    pltpu.sync_copy(z1_hbm.at[pl.ds(s * STRIPE, STRIPE)],
                    deg_sh.at[pl.ds(s * STRIPE, STRIPE)])
    for i in range(8):
        onesb[pl.ds(i * 16, 16)] = jnp.full((16,), 1.0, jnp.float32)
    plsc.subcore_barrier()

    def outer(o, carry):
        base = wid * CPT + o * DSUP
        pltpu.sync_copy(col_hbm.at[pl.ds(base, DSUP)], colb)
        for j in range(DSUP):
            pltpu.async_copy(onesb, deg_sh.at[colb.at[j]], sem, add=True)
        for j in range(DSUP):
            pltpu.make_async_copy(onesb, deg_sh.at[colb.at[j]], sem).wait()
        return carry

    lax.fori_loop(0, DOUTER, outer, 0)
    plsc.subcore_barrier()
    pltpu.sync_copy(deg_sh.at[pl.ds(s * STRIPE, STRIPE)],
                    deg_hbm.at[pl.ds(c * NP + s * STRIPE, STRIPE)])


@functools.partial(
    pl.kernel,
    out_type=jax.ShapeDtypeStruct((2, NP, D), jnp.float32),
    mesh=_mesh,
    scratch_types=[
        pltpu.VMEM_SHARED((NP, D), jnp.float32),
        pltpu.VMEM((4, SUP, 128), jnp.int32),
        pltpu.VMEM((4, SUP, 128), jnp.int32),
        pltpu.VMEM((2, SUP, 128, D), jnp.float32),
        pltpu.SemaphoreType.DMA((4,)),
        pltpu.SemaphoreType.DMA((2,)),
        pltpu.SemaphoreType.DMA((2,)),
    ],
    compiler_params=_sc_params,
)
def _sc_spmm(y_hbm, row_hbm, col_hbm, z2_hbm,
             acc_hbm,
             acc_sh, ibr, ibc, gbuf, sem_i, sem_g, sem_s):
    c = lax.axis_index("c")
    s = lax.axis_index("s")
    wid = s * 2 + c
    base = wid * CPT

    pltpu.sync_copy(z2_hbm.at[pl.ds(s * STRIPE, STRIPE)],
                    acc_sh.at[pl.ds(s * STRIPE, STRIPE)])
    plsc.subcore_barrier()

    def fire_idx(g, d):
        off = base + g * SUP
        pltpu.async_copy(row_hbm.at[pl.ds(off, SUP)], ibr.at[d], sem_i.at[d])
        pltpu.async_copy(col_hbm.at[pl.ds(off, SUP)], ibc.at[d], sem_i.at[d])

    def wait_idx(g, d):
        off = base + g * SUP
        pltpu.make_async_copy(row_hbm.at[pl.ds(off, SUP)], ibr.at[d],
                              sem_i.at[d]).wait()
        pltpu.make_async_copy(col_hbm.at[pl.ds(off, SUP)], ibc.at[d],
                              sem_i.at[d]).wait()

    def fire_g(d, par):
        for j in range(SUP):
            pltpu.async_copy(y_hbm.at[ibr.at[d, j]], gbuf.at[par, j],
                             sem_g.at[par])

    def wait_g(d, par):
        for j in range(SUP):
            pltpu.make_async_copy(y_hbm.at[ibr.at[d, j]], gbuf.at[par, j],
                                  sem_g.at[par]).wait()

    def fire_s(d, par):
        for j in range(SUP):
            pltpu.async_copy(gbuf.at[par, j], acc_sh.at[ibc.at[d, j]],
                             sem_s.at[par], add=True)

    def wait_s(d, par):
        for j in range(SUP):
            pltpu.make_async_copy(gbuf.at[par, j], acc_sh.at[ibc.at[d, j]],
                                  sem_s.at[par]).wait()

    fire_idx(0, 0)
    fire_idx(1, 1)
    wait_idx(0, 0)
    fire_g(0, 0)

    def body(i, carry):
        for k in range(4):
            g = 4 * i + k
            par = k % 2
            q = 1 - par
            wait_g(k, par)
            fire_s(k, par)

            @pl.when(g >= 1)
            def _():
                wait_s((k - 1) % 4, q)

            @pl.when(g + 2 < OUTER)
            def _():
                fire_idx(g + 2, (k + 2) % 4)

            @pl.when(g + 1 < OUTER)
            def _():
                wait_idx(g + 1, (k + 1) % 4)
                fire_g((k + 1) % 4, q)
        return carry

    lax.fori_loop(0, OUTER // 4, body, 0)
    wait_s(3, 1)
    plsc.subcore_barrier()
    pltpu.sync_copy(acc_sh.at[pl.ds(s * STRIPE, STRIPE)],
                    acc_hbm.at[c, pl.ds(s * STRIPE, STRIPE)])


@functools.partial(
    pl.kernel,
    out_type=jax.ShapeDtypeStruct((BATCH * CTX, D), jnp.float32),
    mesh=_mesh,
    scratch_types=[
        pltpu.VMEM((C_PER,), jnp.int32),
        pltpu.VMEM((C_PER, D), jnp.float32),
        pltpu.SemaphoreType.DMA,
    ],
    compiler_params=_sc_params,
)
def _sc_ctx_gather(h2_hbm, cidx_hbm, out_hbm, cidxb, crows, sem):
    c = lax.axis_index("c")
    s = lax.axis_index("s")
    wid = s * 2 + c
    pltpu.sync_copy(cidx_hbm.at[pl.ds(wid * C_PER, C_PER)], cidxb)
    pltpu.async_copy(h2_hbm.at[cidxb], crows, sem).wait()
    pltpu.sync_copy(crows, out_hbm.at[pl.ds(wid * C_PER, C_PER)])


def _tc_prep_body(emb_ref, deg_ref, w4_ref, bexp_ref, y1_ref, dinv_ref):
    d = deg_ref[0, :NR] + deg_ref[1, :NR] + 1.0
    dinv_n = lax.rsqrt(d)
    dinv = lax.dot_general(dinv_n, bexp_ref[...], (((1,), (0,)), ((), ())),
                           preferred_element_type=jnp.float32)
    dinv_ref[...] = dinv
    xw = lax.dot_general(emb_ref[...], w4_ref[...], (((1,), (0,)), ((), ())),
                         preferred_element_type=jnp.float32)
    y1_ref[:NR] = dinv * xw
    y1_ref[NR:] = jnp.zeros((NRP - NR, 128), jnp.float32)


_tc_prep = pl.pallas_call(
    _tc_prep_body,
    out_shape=[
        jax.ShapeDtypeStruct((NRP, 128), jnp.float32),
        jax.ShapeDtypeStruct((NR, 128), jnp.float32),
    ],
)


def _tc_mid_body(acc_ref, y1_ref, dinv_ref, b1_ref, w4_ref, y2_ref):
    z = acc_ref[0, :NR] + acc_ref[1, :NR] + y1_ref[:NR]
    dinv = dinv_ref[...]
    h1 = jnp.maximum(dinv * z + b1_ref[...], 0.0)
    y2 = dinv * lax.dot_general(h1, w4_ref[...], (((1,), (0,)), ((), ())),
                                preferred_element_type=jnp.float32)
    y2_ref[:NR] = y2
    y2_ref[NR:] = jnp.zeros((NRP - NR, 128), jnp.float32)


_tc_mid = pl.pallas_call(
    _tc_mid_body,
    out_shape=jax.ShapeDtypeStruct((NRP, 128), jnp.float32),
)


def _tc_post_body(acc_ref, y2_ref, dinv_ref, b2_ref, h2_ref):
    z = acc_ref[0, :NR] + acc_ref[1, :NR] + y2_ref[:NR]
    h2_ref[...] = dinv_ref[...] * z + b2_ref[...]


_tc_post = pl.pallas_call(
    _tc_post_body,
    out_shape=jax.ShapeDtypeStruct((NR, 128), jnp.float32),
)


SB = 2048


def _tc_fc_body(x_ref, w_ref, b_ref, out_ref):
    out_ref[...] = lax.dot_general(
        x_ref[...], w_ref[...], _NT,
        preferred_element_type=jnp.float32) + b_ref[...]


_tc_fc = pl.pallas_call(
    _tc_fc_body,
    grid=(pl.cdiv(N_SERVICES, SB),),
    in_specs=[
        pl.BlockSpec((BATCH, 128), lambda i: (0, 0)),
        pl.BlockSpec((SB, 128), lambda i: (i, 0)),
        pl.BlockSpec((1, SB), lambda i: (0, i)),
    ],
    out_specs=pl.BlockSpec((BATCH, SB), lambda i: (0, i)),
    out_shape=jax.ShapeDtypeStruct((BATCH, N_SERVICES), jnp.float32),
)


def kernel(user_idx, context_idx, edge_index, user_emb, service_emb,
           gcn1_W, gcn1_b, gcn2_W, gcn2_b, fc_W, fc_b):
    row = edge_index[0].astype(jnp.int32)
    col = edge_index[1].astype(jnp.int32)
    pad = N_SERVICES + (jnp.arange(EPC * 128 - N_EDGES, dtype=jnp.int32)
                        % (NP - N_SERVICES))
    row2 = jnp.concatenate([row, pad]).reshape(EPC, 128)
    col2 = jnp.concatenate([col, pad]).reshape(EPC, 128)
    z1 = jnp.zeros((NP,), jnp.float32)
    z2 = jnp.zeros((NP, D), jnp.float32)

    zblk = jnp.zeros((D, D), jnp.float32)
    w4_1 = jnp.block([[gcn1_W.T if i == j else zblk for j in range(4)]
                      for i in range(4)])
    w4_2 = jnp.block([[gcn2_W.T if i == j else zblk for j in range(4)]
                      for i in range(4)])
    bexp = jnp.repeat(jnp.eye(4, dtype=jnp.float32), D, axis=1)
    b1_4 = jnp.tile(gcn1_b, 4).reshape(1, 128)
    b2_4 = jnp.tile(gcn2_b, 4).reshape(1, 128)

    degf, uvec = _sc_deg_user(col2, user_idx.astype(jnp.int32), user_emb, z1)
    y1p, dinvp = _tc_prep(service_emb.reshape(NR, 128),
                          degf.reshape(2, NRP, 4), w4_1, bexp)
    acc1 = _sc_spmm(y1p.reshape(NP, D), row2, col2, z2)
    y2p = _tc_mid(acc1.reshape(2, NRP, 128), y1p, dinvp, b1_4, w4_2)
    acc2 = _sc_spmm(y2p.reshape(NP, D), row2, col2, z2)
    h2p = _tc_post(acc2.reshape(2, NRP, 128), y2p, dinvp, b2_4)
    ctx = _sc_ctx_gather(h2p.reshape(N_SERVICES, D),
                         context_idx.reshape(-1).astype(jnp.int32))
    x = jnp.concatenate([uvec, ctx.reshape(BATCH, CTX * D)], axis=1)
    return _tc_fc(x, fc_W, fc_b.reshape(1, N_SERVICES))


# speedup vs baseline: 62.2103x; 1.1971x over previous
import functools

import jax
import jax.numpy as jnp
from jax import lax
from jax.experimental import pallas as pl
from jax.experimental.pallas import tpu as pltpu
from jax.experimental.pallas import tpu_sc as plsc

N_SERVICES = 50000
N_USERS = 100000
D = 32
BATCH = 1024
CTX = 3
N_EDGES = 1600000

NP = 50048
NR = 12500
NRP = NP // 4
STRIPE = NP // 16
EPC = 12800
CPT = EPC // 32
GSUP = 8
GOUTER = CPT // GSUP
DSUP = 20
DOUTER = CPT // DSUP
U_PER = BATCH // 32
C_PER = (BATCH * CTX) // 32

_mesh = plsc.VectorSubcoreMesh(core_axis_name="c", subcore_axis_name="s")
_sc_params = pltpu.CompilerParams(use_tc_tiling_on_sc=False)

_NT = (((1,), (1,)), ((), ()))


@functools.partial(
    pl.kernel,
    out_type=[
        jax.ShapeDtypeStruct((2 * NP,), jnp.float32),
        jax.ShapeDtypeStruct((BATCH, D), jnp.float32),
    ],
    mesh=_mesh,
    scratch_types=[
        pltpu.VMEM_SHARED((NP,), jnp.float32),
        pltpu.VMEM((DSUP, 128), jnp.int32),
        pltpu.VMEM((128,), jnp.float32),
        pltpu.VMEM((U_PER,), jnp.int32),
        pltpu.VMEM((U_PER, D), jnp.float32),
        pltpu.SemaphoreType.DMA,
    ],
    compiler_params=_sc_params,
)
def _sc_deg_user(col_hbm, uidx_hbm, uemb_hbm, z1_hbm,
                 deg_hbm, uvec_hbm,
                 deg_sh, colb, onesb, uidxb, urows, sem):
    c = lax.axis_index("c")
    s = lax.axis_index("s")
    wid = s * 2 + c

    pltpu.sync_copy(uidx_hbm.at[pl.ds(wid * U_PER, U_PER)], uidxb)
    pltpu.async_copy(uemb_hbm.at[uidxb], urows, sem).wait()
    pltpu.sync_copy(urows, uvec_hbm.at[pl.ds(wid * U_PER, U_PER)])

    pltpu.sync_copy(z1_hbm.at[pl.ds(s * STRIPE, STRIPE)],
                    deg_sh.at[pl.ds(s * STRIPE, STRIPE)])
    for i in range(8):
        onesb[pl.ds(i * 16, 16)] = jnp.full((16,), 1.0, jnp.float32)
    plsc.subcore_barrier()

    def outer(o, carry):
        base = wid * CPT + o * DSUP
        pltpu.sync_copy(col_hbm.at[pl.ds(base, DSUP)], colb)
        for j in range(DSUP):
            pltpu.async_copy(onesb, deg_sh.at[colb.at[j]], sem, add=True)
        for j in range(DSUP):
            pltpu.make_async_copy(onesb, deg_sh.at[colb.at[j]], sem).wait()
        return carry

    lax.fori_loop(0, DOUTER, outer, 0)
    plsc.subcore_barrier()
    pltpu.sync_copy(deg_sh.at[pl.ds(s * STRIPE, STRIPE)],
                    deg_hbm.at[pl.ds(c * NP + s * STRIPE, STRIPE)])


@functools.partial(
    pl.kernel,
    out_type=jax.ShapeDtypeStruct((2, NP, D), jnp.float32),
    mesh=_mesh,
    scratch_types=[
        pltpu.VMEM_SHARED((NP, D), jnp.float32),
        pltpu.VMEM((2, GSUP, 128), jnp.int32),
        pltpu.VMEM((2, GSUP, 128), jnp.int32),
        pltpu.VMEM((4, 128, D), jnp.float32),
        pltpu.SemaphoreType.DMA((2,)),
        pltpu.SemaphoreType.DMA((4,)),
        pltpu.SemaphoreType.DMA((4,)),
    ],
    compiler_params=_sc_params,
)
def _sc_spmm(y_hbm, row_hbm, col_hbm, z2_hbm,
             acc_hbm,
             acc_sh, ibr, ibc, gbuf, sem_i, sem_g, sem_s):
    c = lax.axis_index("c")
    s = lax.axis_index("s")
    wid = s * 2 + c
    base = wid * CPT

    pltpu.sync_copy(z2_hbm.at[pl.ds(s * STRIPE, STRIPE)],
                    acc_sh.at[pl.ds(s * STRIPE, STRIPE)])
    plsc.subcore_barrier()

    def fire_idx(g, d):
        off = base + g * GSUP
        pltpu.async_copy(row_hbm.at[pl.ds(off, GSUP)], ibr.at[d], sem_i.at[d])
        pltpu.async_copy(col_hbm.at[pl.ds(off, GSUP)], ibc.at[d], sem_i.at[d])

    def wait_idx(g, d):
        off = base + g * GSUP
        pltpu.make_async_copy(row_hbm.at[pl.ds(off, GSUP)], ibr.at[d],
                              sem_i.at[d]).wait()
        pltpu.make_async_copy(col_hbm.at[pl.ds(off, GSUP)], ibc.at[d],
                              sem_i.at[d]).wait()

    def fire_g(d, jrow, r):
        pltpu.async_copy(y_hbm.at[ibr.at[d, jrow]], gbuf.at[r], sem_g.at[r])

    def wait_g(d, jrow, r):
        pltpu.make_async_copy(y_hbm.at[ibr.at[d, jrow]], gbuf.at[r],
                              sem_g.at[r]).wait()

    def fire_s(d, jrow, r):
        pltpu.async_copy(gbuf.at[r], acc_sh.at[ibc.at[d, jrow]],
                         sem_s.at[r], add=True)

    def wait_s(d, jrow, r):
        pltpu.make_async_copy(gbuf.at[r], acc_sh.at[ibc.at[d, jrow]],
                              sem_s.at[r]).wait()

    def dloc(j):
        jj = j % 16
        return (0 if jj < 8 else 1), jj % 8

    fire_idx(0, 0)
    fire_idx(1, 1)

    def body(i, carry):
        for j in range(16):
            r = j % 4
            d_cur, row_cur = dloc(j)

            dm4, rowm4 = dloc(j - 4)
            if j >= 4:
                wait_s(dm4, rowm4, r)
            else:
                @pl.when(i >= 1)
                def _():
                    wait_s(dm4, rowm4, r)

            if j == 0:
                wait_idx(2 * i, 0)
            if j == 8:
                wait_idx(2 * i + 1, 1)

            fire_g(d_cur, row_cur, r)

            if j == 3:
                @pl.when(i >= 1)
                def _():
                    fire_idx(2 * i + 1, 1)
            if j == 11:
                @pl.when(2 * i + 2 < GOUTER)
                def _():
                    fire_idx(2 * i + 2, 0)

            dm2, rowm2 = dloc(j - 2)
            r2 = (j - 2) % 4
            if j >= 2:
                wait_g(dm2, rowm2, r2)
                fire_s(dm2, rowm2, r2)
            else:
                @pl.when(i >= 1)
                def _():
                    wait_g(dm2, rowm2, r2)
                    fire_s(dm2, rowm2, r2)
        return carry

    lax.fori_loop(0, GOUTER // 2, body, 0)

    wait_g(1, 6, 2)
    fire_s(1, 6, 2)
    wait_g(1, 7, 3)
    fire_s(1, 7, 3)
    wait_s(1, 4, 0)
    wait_s(1, 5, 1)
    wait_s(1, 6, 2)
    wait_s(1, 7, 3)
    plsc.subcore_barrier()
    pltpu.sync_copy(acc_sh.at[pl.ds(s * STRIPE, STRIPE)],
                    acc_hbm.at[c, pl.ds(s * STRIPE, STRIPE)])


@functools.partial(
    pl.kernel,
    out_type=jax.ShapeDtypeStruct((BATCH * CTX, D), jnp.float32),
    mesh=_mesh,
    scratch_types=[
        pltpu.VMEM((C_PER,), jnp.int32),
        pltpu.VMEM((C_PER, D), jnp.float32),
        pltpu.SemaphoreType.DMA,
    ],
    compiler_params=_sc_params,
)
def _sc_ctx_gather(h2_hbm, cidx_hbm, out_hbm, cidxb, crows, sem):
    c = lax.axis_index("c")
    s = lax.axis_index("s")
    wid = s * 2 + c
    pltpu.sync_copy(cidx_hbm.at[pl.ds(wid * C_PER, C_PER)], cidxb)
    pltpu.async_copy(h2_hbm.at[cidxb], crows, sem).wait()
    pltpu.sync_copy(crows, out_hbm.at[pl.ds(wid * C_PER, C_PER)])


def _tc_prep_body(emb_ref, deg_ref, w4_ref, bexp_ref, y1_ref, dinv_ref):
    d = deg_ref[0, :NR] + deg_ref[1, :NR] + 1.0
    dinv_n = lax.rsqrt(d)
    dinv = lax.dot_general(dinv_n, bexp_ref[...], (((1,), (0,)), ((), ())),
                           preferred_element_type=jnp.float32)
    dinv_ref[...] = dinv
    xw = lax.dot_general(emb_ref[...], w4_ref[...], (((1,), (0,)), ((), ())),
                         preferred_element_type=jnp.float32)
    y1_ref[:NR] = dinv * xw
    y1_ref[NR:] = jnp.zeros((NRP - NR, 128), jnp.float32)


_tc_prep = pl.pallas_call(
    _tc_prep_body,
    out_shape=[
        jax.ShapeDtypeStruct((NRP, 128), jnp.float32),
        jax.ShapeDtypeStruct((NR, 128), jnp.float32),
    ],
)


def _tc_mid_body(acc_ref, y1_ref, dinv_ref, b1_ref, w4_ref, y2_ref):
    z = acc_ref[0, :NR] + acc_ref[1, :NR] + y1_ref[:NR]
    dinv = dinv_ref[...]
    h1 = jnp.maximum(dinv * z + b1_ref[...], 0.0)
    y2 = dinv * lax.dot_general(h1, w4_ref[...], (((1,), (0,)), ((), ())),
                                preferred_element_type=jnp.float32)
    y2_ref[:NR] = y2
    y2_ref[NR:] = jnp.zeros((NRP - NR, 128), jnp.float32)


_tc_mid = pl.pallas_call(
    _tc_mid_body,
    out_shape=jax.ShapeDtypeStruct((NRP, 128), jnp.float32),
)


def _tc_post_body(acc_ref, y2_ref, dinv_ref, b2_ref, h2_ref):
    z = acc_ref[0, :NR] + acc_ref[1, :NR] + y2_ref[:NR]
    h2_ref[...] = dinv_ref[...] * z + b2_ref[...]


_tc_post = pl.pallas_call(
    _tc_post_body,
    out_shape=jax.ShapeDtypeStruct((NR, 128), jnp.float32),
)


SB = 2048


def _tc_fc_body(x_ref, w_ref, b_ref, out_ref):
    out_ref[...] = lax.dot_general(
        x_ref[...], w_ref[...], _NT,
        preferred_element_type=jnp.float32) + b_ref[...]


_tc_fc = pl.pallas_call(
    _tc_fc_body,
    grid=(pl.cdiv(N_SERVICES, SB),),
    in_specs=[
        pl.BlockSpec((BATCH, 128), lambda i: (0, 0)),
        pl.BlockSpec((SB, 128), lambda i: (i, 0)),
        pl.BlockSpec((1, SB), lambda i: (0, i)),
    ],
    out_specs=pl.BlockSpec((BATCH, SB), lambda i: (0, i)),
    out_shape=jax.ShapeDtypeStruct((BATCH, N_SERVICES), jnp.float32),
)


def kernel(user_idx, context_idx, edge_index, user_emb, service_emb,
           gcn1_W, gcn1_b, gcn2_W, gcn2_b, fc_W, fc_b):
    row = edge_index[0].astype(jnp.int32)
    col = edge_index[1].astype(jnp.int32)
    pad = N_SERVICES + (jnp.arange(EPC * 128 - N_EDGES, dtype=jnp.int32)
                        % (NP - N_SERVICES))
    row2 = jnp.concatenate([row, pad]).reshape(EPC, 128)
    col2 = jnp.concatenate([col, pad]).reshape(EPC, 128)
    z1 = jnp.zeros((NP,), jnp.float32)
    z2 = jnp.zeros((NP, D), jnp.float32)

    zblk = jnp.zeros((D, D), jnp.float32)
    w4_1 = jnp.block([[gcn1_W.T if i == j else zblk for j in range(4)]
                      for i in range(4)])
    w4_2 = jnp.block([[gcn2_W.T if i == j else zblk for j in range(4)]
                      for i in range(4)])
    bexp = jnp.repeat(jnp.eye(4, dtype=jnp.float32), D, axis=1)
    b1_4 = jnp.tile(gcn1_b, 4).reshape(1, 128)
    b2_4 = jnp.tile(gcn2_b, 4).reshape(1, 128)

    degf, uvec = _sc_deg_user(col2, user_idx.astype(jnp.int32), user_emb, z1)
    y1p, dinvp = _tc_prep(service_emb.reshape(NR, 128),
                          degf.reshape(2, NRP, 4), w4_1, bexp)
    acc1 = _sc_spmm(y1p.reshape(NP, D), row2, col2, z2)
    y2p = _tc_mid(acc1.reshape(2, NRP, 128), y1p, dinvp, b1_4, w4_2)
    acc2 = _sc_spmm(y2p.reshape(NP, D), row2, col2, z2)
    h2p = _tc_post(acc2.reshape(2, NRP, 128), y2p, dinvp, b2_4)
    ctx = _sc_ctx_gather(h2p.reshape(N_SERVICES, D),
                         context_idx.reshape(-1).astype(jnp.int32))
    x = jnp.concatenate([uvec, ctx.reshape(BATCH, CTX * D)], axis=1)
    return _tc_fc(x, fc_W, fc_b.reshape(1, N_SERVICES))


# speedup vs baseline: 78.3700x; 1.2598x over previous
import functools

import jax
import jax.numpy as jnp
from jax import lax
from jax.experimental import pallas as pl
from jax.experimental.pallas import tpu as pltpu
from jax.experimental.pallas import tpu_sc as plsc

N_SERVICES = 50000
N_USERS = 100000
D = 32
BATCH = 1024
CTX = 3
N_EDGES = 1600000

NP = 50048
NR = 12500
NRP = NP // 4
STRIPE = NP // 16
EPC = 12800
CPT = EPC // 32
GSUP = 8
GOUTER = CPT // GSUP
DSUP = 20
DOUTER = CPT // DSUP
U_PER = BATCH // 32
C_PER = (BATCH * CTX) // 32

_mesh = plsc.VectorSubcoreMesh(core_axis_name="c", subcore_axis_name="s")
_sc_params = pltpu.CompilerParams(use_tc_tiling_on_sc=False)

_NT = (((1,), (1,)), ((), ()))


@functools.partial(
    pl.kernel,
    out_type=jax.ShapeDtypeStruct((2 * NP,), jnp.float32),
    mesh=_mesh,
    scratch_types=[
        pltpu.VMEM_SHARED((NP,), jnp.float32),
        pltpu.VMEM((DSUP, 128), jnp.int32),
        pltpu.VMEM((128,), jnp.float32),
        pltpu.SemaphoreType.DMA,
    ],
    compiler_params=_sc_params,
)
def _sc_deg(col_hbm, z1_hbm, deg_hbm, deg_sh, colb, onesb, sem):
    c = lax.axis_index("c")
    s = lax.axis_index("s")
    wid = s * 2 + c

    pltpu.sync_copy(z1_hbm.at[pl.ds(s * STRIPE, STRIPE)],
                    deg_sh.at[pl.ds(s * STRIPE, STRIPE)])
    for i in range(8):
        onesb[pl.ds(i * 16, 16)] = jnp.full((16,), 1.0, jnp.float32)
    plsc.subcore_barrier()

    def outer(o, carry):
        base = wid * CPT + o * DSUP
        pltpu.sync_copy(col_hbm.at[pl.ds(base, DSUP)], colb)
        for j in range(DSUP):
            pltpu.async_copy(onesb, deg_sh.at[colb.at[j]], sem, add=True)
        for j in range(DSUP):
            pltpu.make_async_copy(onesb, deg_sh.at[colb.at[j]], sem).wait()
        return carry

    lax.fori_loop(0, DOUTER, outer, 0)
    plsc.subcore_barrier()
    pltpu.sync_copy(deg_sh.at[pl.ds(s * STRIPE, STRIPE)],
                    deg_hbm.at[pl.ds(c * NP + s * STRIPE, STRIPE)])


@functools.partial(
    pl.kernel,
    out_type=jax.ShapeDtypeStruct((2, NP, D), jnp.float32),
    mesh=_mesh,
    scratch_types=[
        pltpu.VMEM_SHARED((NP, D), jnp.float32),
        pltpu.VMEM((2, GSUP, 128), jnp.int32),
        pltpu.VMEM((2, GSUP, 128), jnp.int32),
        pltpu.VMEM((4, 128, D), jnp.float32),
        pltpu.SemaphoreType.DMA((2,)),
        pltpu.SemaphoreType.DMA((4,)),
        pltpu.SemaphoreType.DMA((4,)),
    ],
    compiler_params=_sc_params,
)
def _sc_spmm(y_hbm, row_hbm, col_hbm, z2_hbm,
             acc_hbm,
             acc_sh, ibr, ibc, gbuf, sem_i, sem_g, sem_s):
    c = lax.axis_index("c")
    s = lax.axis_index("s")
    wid = s * 2 + c
    base = wid * CPT

    pltpu.sync_copy(z2_hbm.at[pl.ds(s * STRIPE, STRIPE)],
                    acc_sh.at[pl.ds(s * STRIPE, STRIPE)])
    plsc.subcore_barrier()

    def fire_idx(g, d):
        off = base + g * GSUP
        pltpu.async_copy(row_hbm.at[pl.ds(off, GSUP)], ibr.at[d], sem_i.at[d])
        pltpu.async_copy(col_hbm.at[pl.ds(off, GSUP)], ibc.at[d], sem_i.at[d])

    def wait_idx(g, d):
        off = base + g * GSUP
        pltpu.make_async_copy(row_hbm.at[pl.ds(off, GSUP)], ibr.at[d],
                              sem_i.at[d]).wait()
        pltpu.make_async_copy(col_hbm.at[pl.ds(off, GSUP)], ibc.at[d],
                              sem_i.at[d]).wait()

    def fire_g(d, jrow, r):
        pltpu.async_copy(y_hbm.at[ibr.at[d, jrow]], gbuf.at[r], sem_g.at[r])

    def wait_g(d, jrow, r):
        pltpu.make_async_copy(y_hbm.at[ibr.at[d, jrow]], gbuf.at[r],
                              sem_g.at[r]).wait()

    def fire_s(d, jrow, r):
        pltpu.async_copy(gbuf.at[r], acc_sh.at[ibc.at[d, jrow]],
                         sem_s.at[r], add=True)

    def wait_s(d, jrow, r):
        pltpu.make_async_copy(gbuf.at[r], acc_sh.at[ibc.at[d, jrow]],
                              sem_s.at[r]).wait()

    def dloc(j):
        jj = j % 16
        return (0 if jj < 8 else 1), jj % 8

    fire_idx(0, 0)
    fire_idx(1, 1)

    def body(i, carry):
        for j in range(16):
            r = j % 4
            d_cur, row_cur = dloc(j)

            dm4, rowm4 = dloc(j - 4)
            if j >= 4:
                wait_s(dm4, rowm4, r)
            else:
                @pl.when(i >= 1)
                def _():
                    wait_s(dm4, rowm4, r)

            if j == 0:
                wait_idx(2 * i, 0)
            if j == 8:
                wait_idx(2 * i + 1, 1)

            fire_g(d_cur, row_cur, r)

            if j == 3:
                @pl.when(i >= 1)
                def _():
                    fire_idx(2 * i + 1, 1)
            if j == 11:
                @pl.when(2 * i + 2 < GOUTER)
                def _():
                    fire_idx(2 * i + 2, 0)

            dm2, rowm2 = dloc(j - 2)
            r2 = (j - 2) % 4
            if j >= 2:
                wait_g(dm2, rowm2, r2)
                fire_s(dm2, rowm2, r2)
            else:
                @pl.when(i >= 1)
                def _():
                    wait_g(dm2, rowm2, r2)
                    fire_s(dm2, rowm2, r2)
        return carry

    lax.fori_loop(0, GOUTER // 2, body, 0)

    wait_g(1, 6, 2)
    fire_s(1, 6, 2)
    wait_g(1, 7, 3)
    fire_s(1, 7, 3)
    wait_s(1, 4, 0)
    wait_s(1, 5, 1)
    wait_s(1, 6, 2)
    wait_s(1, 7, 3)
    plsc.subcore_barrier()
    pltpu.sync_copy(acc_sh.at[pl.ds(s * STRIPE, STRIPE)],
                    acc_hbm.at[c, pl.ds(s * STRIPE, STRIPE)])


@functools.partial(
    pl.kernel,
    out_type=[
        jax.ShapeDtypeStruct((BATCH * CTX, D), jnp.float32),
        jax.ShapeDtypeStruct((BATCH, D), jnp.float32),
    ],
    mesh=_mesh,
    scratch_types=[
        pltpu.VMEM((C_PER,), jnp.int32),
        pltpu.VMEM((C_PER, D), jnp.float32),
        pltpu.VMEM((U_PER,), jnp.int32),
        pltpu.VMEM((U_PER, D), jnp.float32),
        pltpu.SemaphoreType.DMA,
        pltpu.SemaphoreType.DMA,
    ],
    compiler_params=_sc_params,
)
def _sc_gathers(h2_hbm, cidx_hbm, uemb_hbm, uidx_hbm,
                ctx_hbm, uvec_hbm,
                cidxb, crows, uidxb, urows, sem_c, sem_u):
    c = lax.axis_index("c")
    s = lax.axis_index("s")
    wid = s * 2 + c
    pltpu.sync_copy(cidx_hbm.at[pl.ds(wid * C_PER, C_PER)], cidxb)
    pltpu.sync_copy(uidx_hbm.at[pl.ds(wid * U_PER, U_PER)], uidxb)
    pltpu.async_copy(h2_hbm.at[cidxb], crows, sem_c)
    pltpu.async_copy(uemb_hbm.at[uidxb], urows, sem_u)
    pltpu.make_async_copy(h2_hbm.at[cidxb], crows, sem_c).wait()
    pltpu.make_async_copy(uemb_hbm.at[uidxb], urows, sem_u).wait()
    pltpu.sync_copy(crows, ctx_hbm.at[pl.ds(wid * C_PER, C_PER)])
    pltpu.sync_copy(urows, uvec_hbm.at[pl.ds(wid * U_PER, U_PER)])


def _tc_prep_body(emb_ref, deg_ref, w4_ref, bexp_ref, y1_ref, dinv_ref):
    d = deg_ref[0, :NR] + deg_ref[1, :NR] + 1.0
    dinv_n = lax.rsqrt(d)
    dinv = lax.dot_general(dinv_n, bexp_ref[...], (((1,), (0,)), ((), ())),
                           preferred_element_type=jnp.float32)
    dinv_ref[...] = dinv
    xw = lax.dot_general(emb_ref[...], w4_ref[...], (((1,), (0,)), ((), ())),
                         preferred_element_type=jnp.float32)
    y1_ref[:NR] = dinv * xw
    y1_ref[NR:] = jnp.zeros((NRP - NR, 128), jnp.float32)


_tc_prep = pl.pallas_call(
    _tc_prep_body,
    out_shape=[
        jax.ShapeDtypeStruct((NRP, 128), jnp.float32),
        jax.ShapeDtypeStruct((NR, 128), jnp.float32),
    ],
)


def _tc_mid_body(acc_ref, y1_ref, dinv_ref, b1_ref, w4_ref, y2_ref):
    z = acc_ref[0, :NR] + acc_ref[1, :NR] + y1_ref[:NR]
    dinv = dinv_ref[...]
    h1 = jnp.maximum(dinv * z + b1_ref[...], 0.0)
    y2 = dinv * lax.dot_general(h1, w4_ref[...], (((1,), (0,)), ((), ())),
                                preferred_element_type=jnp.float32)
    y2_ref[:NR] = y2
    y2_ref[NR:] = jnp.zeros((NRP - NR, 128), jnp.float32)


_tc_mid = pl.pallas_call(
    _tc_mid_body,
    out_shape=jax.ShapeDtypeStruct((NRP, 128), jnp.float32),
)


def _tc_post_body(acc_ref, y2_ref, dinv_ref, b2_ref, h2_ref):
    z = acc_ref[0, :NR] + acc_ref[1, :NR] + y2_ref[:NR]
    h2_ref[...] = dinv_ref[...] * z + b2_ref[...]


_tc_post = pl.pallas_call(
    _tc_post_body,
    out_shape=jax.ShapeDtypeStruct((NR, 128), jnp.float32),
)


SB = 2048


def _tc_fc_body(x_ref, w_ref, b_ref, out_ref):
    out_ref[...] = lax.dot_general(
        w_ref[...], x_ref[...], _NT,
        preferred_element_type=jnp.float32) + b_ref[...]


_tc_fc = pl.pallas_call(
    _tc_fc_body,
    grid=(pl.cdiv(N_SERVICES, SB),),
    in_specs=[
        pl.BlockSpec((BATCH, 128), lambda i: (0, 0)),
        pl.BlockSpec((SB, 128), lambda i: (i, 0)),
        pl.BlockSpec((SB, 1), lambda i: (i, 0)),
    ],
    out_specs=pl.BlockSpec((SB, BATCH), lambda i: (i, 0)),
    out_shape=jax.ShapeDtypeStruct((N_SERVICES, BATCH), jnp.float32),
)


def kernel(user_idx, context_idx, edge_index, user_emb, service_emb,
           gcn1_W, gcn1_b, gcn2_W, gcn2_b, fc_W, fc_b):
    row = edge_index[0].astype(jnp.int32)
    col = edge_index[1].astype(jnp.int32)
    pad = N_SERVICES + (jnp.arange(EPC * 128 - N_EDGES, dtype=jnp.int32)
                        % (NP - N_SERVICES))
    row2 = jnp.concatenate([row, pad]).reshape(EPC, 128)
    col2 = jnp.concatenate([col, pad]).reshape(EPC, 128)
    z1 = jnp.zeros((NP,), jnp.float32)
    z2 = jnp.zeros((NP, D), jnp.float32)

    zblk = jnp.zeros((D, D), jnp.float32)
    w4_1 = jnp.block([[gcn1_W.T if i == j else zblk for j in range(4)]
                      for i in range(4)])
    w4_2 = jnp.block([[gcn2_W.T if i == j else zblk for j in range(4)]
                      for i in range(4)])
    bexp = jnp.repeat(jnp.eye(4, dtype=jnp.float32), D, axis=1)
    b1_4 = jnp.tile(gcn1_b, 4).reshape(1, 128)
    b2_4 = jnp.tile(gcn2_b, 4).reshape(1, 128)

    degf = _sc_deg(col2, z1)
    y1p, dinvp = _tc_prep(service_emb.reshape(NR, 128),
                          degf.reshape(2, NRP, 4), w4_1, bexp)
    acc1 = _sc_spmm(y1p.reshape(NP, D), row2, col2, z2)
    y2p = _tc_mid(acc1.reshape(2, NRP, 128), y1p, dinvp, b1_4, w4_2)
    acc2 = _sc_spmm(y2p.reshape(NP, D), row2, col2, z2)
    h2p = _tc_post(acc2.reshape(2, NRP, 128), y2p, dinvp, b2_4)
    ctx, uvec = _sc_gathers(h2p.reshape(N_SERVICES, D),
                            context_idx.reshape(-1).astype(jnp.int32),
                            user_emb, user_idx.astype(jnp.int32))
    x = jnp.concatenate([uvec, ctx.reshape(BATCH, CTX * D)], axis=1)
    outT = _tc_fc(x, fc_W, fc_b.reshape(N_SERVICES, 1))
    return outT.T


# speedup vs baseline: 82.0642x; 1.0471x over previous
import functools

import jax
import jax.numpy as jnp
from jax import lax
from jax.experimental import pallas as pl
from jax.experimental.pallas import tpu as pltpu
from jax.experimental.pallas import tpu_sc as plsc

N_SERVICES = 50000
N_USERS = 100000
D = 32
BATCH = 1024
CTX = 3
N_EDGES = 1600000

NP = 50048
NR = 12500
NRP = NP // 4
STRIPE = NP // 16
EW = 125
EPC = 12800
CPT = EPC // 32
GSUP = 8
GOUTER = CPT // GSUP
DSUP = 20
DOUTER = CPT // DSUP
U_PER = BATCH // 32
C_PER = (BATCH * CTX) // 32

_mesh = plsc.VectorSubcoreMesh(core_axis_name="c", subcore_axis_name="s")
_sc_params = pltpu.CompilerParams(use_tc_tiling_on_sc=False)

_NT = (((1,), (1,)), ((), ()))


@functools.partial(
    pl.kernel,
    out_type=jax.ShapeDtypeStruct((2 * NP,), jnp.float32),
    mesh=_mesh,
    scratch_types=[
        pltpu.VMEM_SHARED((NP,), jnp.float32),
        pltpu.VMEM((DSUP, EW), jnp.int32),
        pltpu.VMEM((128,), jnp.float32),
        pltpu.SemaphoreType.DMA,
    ],
    compiler_params=_sc_params,
)
def _sc_deg(eif_hbm, z1_hbm, deg_hbm, deg_sh, colb, onesb, sem):
    c = lax.axis_index("c")
    s = lax.axis_index("s")
    wid = s * 2 + c

    pltpu.sync_copy(z1_hbm.at[pl.ds(s * STRIPE, STRIPE)],
                    deg_sh.at[pl.ds(s * STRIPE, STRIPE)])
    for i in range(8):
        onesb[pl.ds(i * 16, 16)] = jnp.full((16,), 1.0, jnp.float32)
    plsc.subcore_barrier()

    ones = onesb.at[pl.ds(0, EW)]

    def outer(o, carry):
        base = EPC + wid * CPT + o * DSUP
        pltpu.sync_copy(eif_hbm.at[pl.ds(base, DSUP)], colb)
        for j in range(DSUP):
            pltpu.async_copy(ones, deg_sh.at[colb.at[j]], sem, add=True)
        for j in range(DSUP):
            pltpu.make_async_copy(ones, deg_sh.at[colb.at[j]], sem).wait()
        return carry

    lax.fori_loop(0, DOUTER, outer, 0)
    plsc.subcore_barrier()
    pltpu.sync_copy(deg_sh.at[pl.ds(s * STRIPE, STRIPE)],
                    deg_hbm.at[pl.ds(c * NP + s * STRIPE, STRIPE)])


@functools.partial(
    pl.kernel,
    out_type=jax.ShapeDtypeStruct((2, NP, D), jnp.float32),
    mesh=_mesh,
    scratch_types=[
        pltpu.VMEM_SHARED((NP, D), jnp.float32),
        pltpu.VMEM((2, GSUP, EW), jnp.int32),
        pltpu.VMEM((2, GSUP, EW), jnp.int32),
        pltpu.VMEM((4, EW, D), jnp.float32),
        pltpu.SemaphoreType.DMA((2,)),
        pltpu.SemaphoreType.DMA((4,)),
        pltpu.SemaphoreType.DMA((4,)),
    ],
    compiler_params=_sc_params,
)
def _sc_spmm(y_hbm, eif_hbm, z2_hbm,
             acc_hbm,
             acc_sh, ibr, ibc, gbuf, sem_i, sem_g, sem_s):
    c = lax.axis_index("c")
    s = lax.axis_index("s")
    wid = s * 2 + c
    base = wid * CPT

    pltpu.sync_copy(z2_hbm.at[pl.ds(s * STRIPE, STRIPE)],
                    acc_sh.at[pl.ds(s * STRIPE, STRIPE)])
    plsc.subcore_barrier()

    def fire_idx(g, d):
        off = base + g * GSUP
        pltpu.async_copy(eif_hbm.at[pl.ds(off, GSUP)], ibr.at[d], sem_i.at[d])
        pltpu.async_copy(eif_hbm.at[pl.ds(EPC + off, GSUP)], ibc.at[d],
                         sem_i.at[d])

    def wait_idx(g, d):
        off = base + g * GSUP
        pltpu.make_async_copy(eif_hbm.at[pl.ds(off, GSUP)], ibr.at[d],
                              sem_i.at[d]).wait()
        pltpu.make_async_copy(eif_hbm.at[pl.ds(EPC + off, GSUP)], ibc.at[d],
                              sem_i.at[d]).wait()

    def fire_g(d, jrow, r):
        pltpu.async_copy(y_hbm.at[ibr.at[d, jrow]], gbuf.at[r], sem_g.at[r])

    def wait_g(d, jrow, r):
        pltpu.make_async_copy(y_hbm.at[ibr.at[d, jrow]], gbuf.at[r],
                              sem_g.at[r]).wait()

    def fire_s(d, jrow, r):
        pltpu.async_copy(gbuf.at[r], acc_sh.at[ibc.at[d, jrow]],
                         sem_s.at[r], add=True)

    def wait_s(d, jrow, r):
        pltpu.make_async_copy(gbuf.at[r], acc_sh.at[ibc.at[d, jrow]],
                              sem_s.at[r]).wait()

    def dloc(j):
        jj = j % 16
        return (0 if jj < 8 else 1), jj % 8

    fire_idx(0, 0)
    fire_idx(1, 1)

    def body(i, carry):
        for j in range(16):
            r = j % 4
            d_cur, row_cur = dloc(j)

            dm4, rowm4 = dloc(j - 4)
            if j >= 4:
                wait_s(dm4, rowm4, r)
            else:
                @pl.when(i >= 1)
                def _():
                    wait_s(dm4, rowm4, r)

            if j == 0:
                wait_idx(2 * i, 0)
            if j == 8:
                wait_idx(2 * i + 1, 1)

            fire_g(d_cur, row_cur, r)

            if j == 3:
                @pl.when(i >= 1)
                def _():
                    fire_idx(2 * i + 1, 1)
            if j == 11:
                @pl.when(2 * i + 2 < GOUTER)
                def _():
                    fire_idx(2 * i + 2, 0)

            dm2, rowm2 = dloc(j - 2)
            r2 = (j - 2) % 4
            if j >= 2:
                wait_g(dm2, rowm2, r2)
                fire_s(dm2, rowm2, r2)
            else:
                @pl.when(i >= 1)
                def _():
                    wait_g(dm2, rowm2, r2)
                    fire_s(dm2, rowm2, r2)
        return carry

    lax.fori_loop(0, GOUTER // 2, body, 0)

    wait_g(1, 6, 2)
    fire_s(1, 6, 2)
    wait_g(1, 7, 3)
    fire_s(1, 7, 3)
    wait_s(1, 4, 0)
    wait_s(1, 5, 1)
    wait_s(1, 6, 2)
    wait_s(1, 7, 3)
    plsc.subcore_barrier()
    pltpu.sync_copy(acc_sh.at[pl.ds(s * STRIPE, STRIPE)],
                    acc_hbm.at[c, pl.ds(s * STRIPE, STRIPE)])


@functools.partial(
    pl.kernel,
    out_type=[
        jax.ShapeDtypeStruct((BATCH * CTX, D), jnp.float32),
        jax.ShapeDtypeStruct((BATCH, D), jnp.float32),
    ],
    mesh=_mesh,
    scratch_types=[
        pltpu.VMEM((C_PER,), jnp.int32),
        pltpu.VMEM((C_PER, D), jnp.float32),
        pltpu.VMEM((U_PER,), jnp.int32),
        pltpu.VMEM((U_PER, D), jnp.float32),
        pltpu.SemaphoreType.DMA,
        pltpu.SemaphoreType.DMA,
    ],
    compiler_params=_sc_params,
)
def _sc_gathers(h2_hbm, cidx_hbm, uemb_hbm, uidx_hbm,
                ctx_hbm, uvec_hbm,
                cidxb, crows, uidxb, urows, sem_c, sem_u):
    c = lax.axis_index("c")
    s = lax.axis_index("s")
    wid = s * 2 + c
    pltpu.sync_copy(cidx_hbm.at[pl.ds(wid * C_PER, C_PER)], cidxb)
    pltpu.sync_copy(uidx_hbm.at[pl.ds(wid * U_PER, U_PER)], uidxb)
    pltpu.async_copy(h2_hbm.at[cidxb], crows, sem_c)
    pltpu.async_copy(uemb_hbm.at[uidxb], urows, sem_u)
    pltpu.make_async_copy(h2_hbm.at[cidxb], crows, sem_c).wait()
    pltpu.make_async_copy(uemb_hbm.at[uidxb], urows, sem_u).wait()
    pltpu.sync_copy(crows, ctx_hbm.at[pl.ds(wid * C_PER, C_PER)])
    pltpu.sync_copy(urows, uvec_hbm.at[pl.ds(wid * U_PER, U_PER)])


def _tc_prep_body(emb_ref, deg_ref, w4_ref, bexp_ref, y1_ref, dinv_ref):
    d = deg_ref[0, :NR] + deg_ref[1, :NR] + 1.0
    dinv_n = lax.rsqrt(d)
    dinv = lax.dot_general(dinv_n, bexp_ref[...], (((1,), (0,)), ((), ())),
                           preferred_element_type=jnp.float32)
    dinv_ref[...] = dinv
    xw = lax.dot_general(emb_ref[...], w4_ref[...], (((1,), (0,)), ((), ())),
                         preferred_element_type=jnp.float32)
    y1_ref[:NR] = dinv * xw
    y1_ref[NR:] = jnp.zeros((NRP - NR, 128), jnp.float32)


_tc_prep = pl.pallas_call(
    _tc_prep_body,
    out_shape=[
        jax.ShapeDtypeStruct((NRP, 128), jnp.float32),
        jax.ShapeDtypeStruct((NR, 128), jnp.float32),
    ],
)


def _tc_mid_body(acc_ref, y1_ref, dinv_ref, b1_ref, w4_ref, y2_ref):
    z = acc_ref[0, :NR] + acc_ref[1, :NR] + y1_ref[:NR]
    dinv = dinv_ref[...]
    h1 = jnp.maximum(dinv * z + b1_ref[...], 0.0)
    y2 = dinv * lax.dot_general(h1, w4_ref[...], (((1,), (0,)), ((), ())),
                                preferred_element_type=jnp.float32)
    y2_ref[:NR] = y2
    y2_ref[NR:] = jnp.zeros((NRP - NR, 128), jnp.float32)


_tc_mid = pl.pallas_call(
    _tc_mid_body,
    out_shape=jax.ShapeDtypeStruct((NRP, 128), jnp.float32),
)


def _tc_post_body(acc_ref, y2_ref, dinv_ref, b2_ref, h2_ref):
    z = acc_ref[0, :NR] + acc_ref[1, :NR] + y2_ref[:NR]
    h2_ref[...] = dinv_ref[...] * z + b2_ref[...]


_tc_post = pl.pallas_call(
    _tc_post_body,
    out_shape=jax.ShapeDtypeStruct((NR, 128), jnp.float32),
)


SB = 2048


def _tc_fc_body(x_ref, w_ref, b_ref, out_ref):
    out_ref[...] = lax.dot_general(
        w_ref[...], x_ref[...], _NT,
        preferred_element_type=jnp.float32) + b_ref[...]


_tc_fc = pl.pallas_call(
    _tc_fc_body,
    grid=(pl.cdiv(N_SERVICES, SB),),
    in_specs=[
        pl.BlockSpec((BATCH, 128), lambda i: (0, 0)),
        pl.BlockSpec((SB, 128), lambda i: (i, 0)),
        pl.BlockSpec((SB, 1), lambda i: (i, 0)),
    ],
    out_specs=pl.BlockSpec((SB, BATCH), lambda i: (i, 0)),
    out_shape=jax.ShapeDtypeStruct((N_SERVICES, BATCH), jnp.float32),
)


def kernel(user_idx, context_idx, edge_index, user_emb, service_emb,
           gcn1_W, gcn1_b, gcn2_W, gcn2_b, fc_W, fc_b):
    ei2 = edge_index.astype(jnp.int32).reshape(2 * EPC, EW)
    z1 = jnp.zeros((NP,), jnp.float32)
    z2 = jnp.zeros((NP, D), jnp.float32)

    zblk = jnp.zeros((D, D), jnp.float32)
    w4_1 = jnp.block([[gcn1_W.T if i == j else zblk for j in range(4)]
                      for i in range(4)])
    w4_2 = jnp.block([[gcn2_W.T if i == j else zblk for j in range(4)]
                      for i in range(4)])
    bexp = jnp.repeat(jnp.eye(4, dtype=jnp.float32), D, axis=1)
    b1_4 = jnp.tile(gcn1_b, 4).reshape(1, 128)
    b2_4 = jnp.tile(gcn2_b, 4).reshape(1, 128)

    degf = _sc_deg(ei2, z1)
    y1p, dinvp = _tc_prep(service_emb.reshape(NR, 128),
                          degf.reshape(2, NRP, 4), w4_1, bexp)
    acc1 = _sc_spmm(y1p.reshape(NP, D), ei2, z2)
    y2p = _tc_mid(acc1.reshape(2, NRP, 128), y1p, dinvp, b1_4, w4_2)
    acc2 = _sc_spmm(y2p.reshape(NP, D), ei2, z2)
    h2p = _tc_post(acc2.reshape(2, NRP, 128), y2p, dinvp, b2_4)
    ctx, uvec = _sc_gathers(h2p.reshape(N_SERVICES, D),
                            context_idx.reshape(-1).astype(jnp.int32),
                            user_emb, user_idx.astype(jnp.int32))
    x = jnp.concatenate([uvec, ctx.reshape(BATCH, CTX * D)], axis=1)
    outT = _tc_fc(x, fc_W, fc_b.reshape(N_SERVICES, 1))
    return outT.T
